# Initial kernel scaffold; baseline (speedup 1.0000x reference)
import functools

import jax
import jax.numpy as jnp
from jax import lax
from jax.experimental import pallas as pl
from jax.experimental.pallas import tpu as pltpu
from jax.experimental.pallas import tpu_sc as plsc

N = 10000
E = 160000
NC = 2
NS = 16
ROWS_PER_TILE = N // NS
CHUNK = 128

EDGES_PER_TILE_AGG = E // NS
AGG_FULL_CHUNKS = EDGES_PER_TILE_AGG // CHUNK
AGG_TAIL = EDGES_PER_TILE_AGG - AGG_FULL_CHUNKS * CHUNK

EDGES_PER_TILE_DEG = E // (NC * NS)
DEG_FULL_CHUNKS = EDGES_PER_TILE_DEG // CHUNK
DEG_TAIL = EDGES_PER_TILE_DEG - DEG_FULL_CHUNKS * CHUNK


def _mesh():
    return plsc.VectorSubcoreMesh(core_axis_name="c", subcore_axis_name="s")


@functools.partial(
    pl.kernel,
    out_type=jax.ShapeDtypeStruct((NC, N, 16), jnp.float32),
    mesh=_mesh(),
    scratch_types=[
        pltpu.VMEM((CHUNK,), jnp.int32),
        pltpu.VMEM((DEG_TAIL,), jnp.int32),
        pltpu.VMEM((CHUNK, 16), jnp.float32),
        pltpu.VMEM((ROWS_PER_TILE, 16), jnp.float32),
        pltpu.VMEM_SHARED((N, 16), jnp.float32),
    ],
)
def _sc_degree(dst_hbm, out_hbm, dstv, dstv_t, onesv, zerov, acc):
    c = lax.axis_index("c")
    s = lax.axis_index("s")

    @pl.loop(0, CHUNK)
    def _(i):
        onesv[i, :] = jnp.ones((16,), jnp.float32)

    @pl.loop(0, ROWS_PER_TILE)
    def _(i):
        zerov[i, :] = jnp.zeros((16,), jnp.float32)

    r0 = s * ROWS_PER_TILE
    pltpu.sync_copy(zerov, acc.at[pl.ds(r0, ROWS_PER_TILE)])
    plsc.subcore_barrier()

    base = c * (E // NC) + s * EDGES_PER_TILE_DEG

    @pl.loop(0, DEG_FULL_CHUNKS)
    def _(j):
        b = pl.multiple_of(base + j * CHUNK, 8)
        pltpu.sync_copy(dst_hbm.at[pl.ds(b, CHUNK)], dstv)
        pltpu.sync_copy(onesv, acc.at[dstv], add=True)

    bt = pl.multiple_of(base + DEG_FULL_CHUNKS * CHUNK, 8)
    pltpu.sync_copy(dst_hbm.at[pl.ds(bt, DEG_TAIL)], dstv_t)
    pltpu.sync_copy(onesv.at[pl.ds(0, DEG_TAIL)], acc.at[dstv_t], add=True)

    plsc.subcore_barrier()
    pltpu.sync_copy(acc.at[pl.ds(r0, ROWS_PER_TILE)],
                    out_hbm.at[c, pl.ds(r0, ROWS_PER_TILE)])


def _make_sc_agg(HD):
    @functools.partial(
        pl.kernel,
        out_type=jax.ShapeDtypeStruct((2 * N, HD), jnp.float32),
        mesh=_mesh(),
        scratch_types=[
            pltpu.VMEM((CHUNK,), jnp.int32),
            pltpu.VMEM((CHUNK,), jnp.int32),
            pltpu.VMEM((CHUNK,), jnp.int32),
            pltpu.VMEM((AGG_TAIL,), jnp.int32),
            pltpu.VMEM((AGG_TAIL,), jnp.int32),
            pltpu.VMEM((AGG_TAIL,), jnp.int32),
            pltpu.VMEM((CHUNK, HD), jnp.float32),
            pltpu.VMEM((AGG_TAIL, HD), jnp.float32),
            pltpu.VMEM_SHARED((N, HD), jnp.float32),
            pltpu.SemaphoreType.DMA,
        ],
    )
    def agg(h_hbm, src_hbm, dst_hbm, out_hbm,
            srcv, dstv, idxv, srcv_t, dstv_t, idxv_t, rowsv, rowsv_t,
            acc, sem):
        c = lax.axis_index("c")
        s = lax.axis_index("s")
        r0 = s * ROWS_PER_TILE
        row_off = c * N

        pltpu.sync_copy(h_hbm.at[pl.ds(row_off + r0, ROWS_PER_TILE)],
                        acc.at[pl.ds(r0, ROWS_PER_TILE)])
        plsc.subcore_barrier()

        ebase = s * EDGES_PER_TILE_AGG

        @pl.loop(0, AGG_FULL_CHUNKS)
        def _(j):
            b = pl.multiple_of(ebase + j * CHUNK, 8)
            pltpu.sync_copy(src_hbm.at[pl.ds(b, CHUNK)], srcv)
            pltpu.sync_copy(dst_hbm.at[pl.ds(b, CHUNK)], dstv)
            for i in range(CHUNK // 16):
                idxv[pl.ds(i * 16, 16)] = srcv[pl.ds(i * 16, 16)] + row_off
            pltpu.async_copy(h_hbm.at[idxv], rowsv, sem).wait()
            pltpu.sync_copy(rowsv, acc.at[dstv], add=True)

        bt = pl.multiple_of(ebase + AGG_FULL_CHUNKS * CHUNK, 8)
        pltpu.sync_copy(src_hbm.at[pl.ds(bt, AGG_TAIL)], srcv_t)
        pltpu.sync_copy(dst_hbm.at[pl.ds(bt, AGG_TAIL)], dstv_t)
        idxv_t[...] = srcv_t[...] + row_off
        pltpu.async_copy(h_hbm.at[idxv_t], rowsv_t, sem).wait()
        pltpu.sync_copy(rowsv_t, acc.at[dstv_t], add=True)

        plsc.subcore_barrier()
        pltpu.sync_copy(acc.at[pl.ds(r0, ROWS_PER_TILE)],
                        out_hbm.at[pl.ds(row_off + r0, ROWS_PER_TILE)])

    return agg


_sc_agg128 = _make_sc_agg(128)
_sc_agg32 = _make_sc_agg(32)


def _dinv(degp_ref):
    deg = degp_ref[0, :, 0:1] + degp_ref[1, :, 0:1] + 1.0
    return lax.rsqrt(deg)


def _tc1_body(degp_ref, x_ref, w_ref, out_ref):
    dinv = _dinv(degp_ref)
    h = jnp.dot(x_ref[...], w_ref[...], preferred_element_type=jnp.float32)
    h = h * dinv
    hw = h.shape[1] // 2
    out_ref[0:N, :] = h[:, 0:hw]
    out_ref[N:2 * N, :] = h[:, hw:]


def _tc_mid_body(degp_ref, agg_ref, b_ref, g_ref, be_ref, w_ref, out_ref):
    dinv = _dinv(degp_ref)
    z = jnp.concatenate([agg_ref[0:N, :], agg_ref[N:2 * N, :]], axis=1)
    z = z * dinv + b_ref[...]
    z = jnp.maximum(z, 0.0)
    mean = jnp.mean(z, axis=0, keepdims=True)
    var = jnp.mean((z - mean) * (z - mean), axis=0, keepdims=True)
    z = g_ref[...] * (z - mean) * lax.rsqrt(var + 1e-5) + be_ref[...]
    h = jnp.dot(z, w_ref[...], preferred_element_type=jnp.float32)
    h = h * dinv
    hw = h.shape[1] // 2
    out_ref[0:N, :] = h[:, 0:hw]
    out_ref[N:2 * N, :] = h[:, hw:]


def _tc_final_body(degp_ref, agg_ref, b_ref, out_ref):
    dinv = _dinv(degp_ref)
    z = jnp.concatenate([agg_ref[0:N, :], agg_ref[N:2 * N, :]], axis=1)
    z = z * dinv + b_ref[...]
    m = jnp.max(z, axis=1, keepdims=True)
    zm = z - m
    lse = jnp.log(jnp.sum(jnp.exp(zm), axis=1, keepdims=True))
    out_ref[...] = zm - lse


def _tc_call(body, out_shape, *args):
    return pl.pallas_call(
        body, out_shape=jax.ShapeDtypeStruct(out_shape, jnp.float32))(*args)


def kernel(x, edge_index, W1, b1, W2, b2, W3, b3, gamma1, beta1, gamma2, beta2):
    src = edge_index[0]
    dst = edge_index[1]

    degp = _sc_degree(dst)
    h1 = _tc_call(_tc1_body, (2 * N, 128), degp, x, W1)
    a1 = _sc_agg128(h1, src, dst)
    h2 = _tc_call(_tc_mid_body, (2 * N, 128), degp, a1,
                  b1.reshape(1, -1), gamma1.reshape(1, -1),
                  beta1.reshape(1, -1), W2)
    a2 = _sc_agg128(h2, src, dst)
    h3 = _tc_call(_tc_mid_body, (2 * N, 32), degp, a2,
                  b2.reshape(1, -1), gamma2.reshape(1, -1),
                  beta2.reshape(1, -1), W3)
    a3 = _sc_agg32(h3, src, dst)
    out = _tc_call(_tc_final_body, (N, 64), degp, a3, b3.reshape(1, -1))
    return out


# speedup vs baseline: 9.0453x; 9.0453x over previous
import functools

import jax
import jax.numpy as jnp
from jax import lax
from jax.experimental import pallas as pl
from jax.experimental.pallas import tpu as pltpu
from jax.experimental.pallas import tpu_sc as plsc

N = 10000
NP = 10240
E = 160000
NC = 2
NS = 16
ROWS_PER_TILE = NP // NS
CHUNK = 128

EDGES_PER_TILE_AGG = E // NS
AGG_FULL_CHUNKS = EDGES_PER_TILE_AGG // CHUNK
AGG_TAIL = EDGES_PER_TILE_AGG - AGG_FULL_CHUNKS * CHUNK

EDGES_PER_TILE_DEG = E // (NC * NS)
DEG_FULL_CHUNKS = EDGES_PER_TILE_DEG // CHUNK
DEG_TAIL = EDGES_PER_TILE_DEG - DEG_FULL_CHUNKS * CHUNK


def _mesh():
    return plsc.VectorSubcoreMesh(core_axis_name="c", subcore_axis_name="s")


@functools.partial(
    pl.kernel,
    out_type=jax.ShapeDtypeStruct((NC, NP, 16), jnp.float32),
    mesh=_mesh(),
    scratch_types=[
        pltpu.VMEM((CHUNK,), jnp.int32),
        pltpu.VMEM((DEG_TAIL,), jnp.int32),
        pltpu.VMEM((CHUNK, 16), jnp.float32),
        pltpu.VMEM((ROWS_PER_TILE, 16), jnp.float32),
        pltpu.VMEM_SHARED((NP, 16), jnp.float32),
    ],
    compiler_params=pltpu.CompilerParams(use_tc_tiling_on_sc=False),
)
def _sc_degree(dst_hbm, ones_hbm, zeros_hbm, out_hbm, dstv, dstv_t, onesv, zerov, acc):
    c = lax.axis_index("c")
    s = lax.axis_index("s")

    pltpu.sync_copy(ones_hbm, onesv)
    pltpu.sync_copy(zeros_hbm, zerov)

    r0 = s * ROWS_PER_TILE
    pltpu.sync_copy(zerov, acc.at[pl.ds(r0, ROWS_PER_TILE)])
    plsc.subcore_barrier()

    base = c * (E // NC) + s * EDGES_PER_TILE_DEG

    if True:
        @pl.loop(0, DEG_FULL_CHUNKS)
        def _(j):
            b = pl.multiple_of(base + j * CHUNK, 8)
            pltpu.sync_copy(dst_hbm.at[pl.ds(b, CHUNK)], dstv)
            pltpu.sync_copy(onesv, acc.at[dstv], add=True)

        bt = pl.multiple_of(base + DEG_FULL_CHUNKS * CHUNK, 8)
        pltpu.sync_copy(dst_hbm.at[pl.ds(bt, DEG_TAIL)], dstv_t)
        pltpu.sync_copy(onesv.at[pl.ds(0, DEG_TAIL)], acc.at[dstv_t], add=True)

    plsc.subcore_barrier()
    pltpu.sync_copy(acc.at[pl.ds(r0, ROWS_PER_TILE)],
                    out_hbm.at[c, pl.ds(r0, ROWS_PER_TILE)])


def _make_sc_agg(HD):
    @functools.partial(
        pl.kernel,
        out_type=jax.ShapeDtypeStruct((2 * NP, HD), jnp.float32),
        mesh=_mesh(),
        scratch_types=[
            pltpu.VMEM((CHUNK,), jnp.int32),
            pltpu.VMEM((CHUNK,), jnp.int32),
            pltpu.VMEM((CHUNK,), jnp.int32),
            pltpu.VMEM((AGG_TAIL,), jnp.int32),
            pltpu.VMEM((AGG_TAIL,), jnp.int32),
            pltpu.VMEM((AGG_TAIL,), jnp.int32),
            pltpu.VMEM((CHUNK, HD), jnp.float32),
            pltpu.VMEM((AGG_TAIL, HD), jnp.float32),
            pltpu.VMEM_SHARED((NP, HD), jnp.float32),
            pltpu.SemaphoreType.DMA,
        ],
        compiler_params=pltpu.CompilerParams(use_tc_tiling_on_sc=False),
    )
    def agg(h_hbm, src_hbm, dst_hbm, out_hbm,
            srcv, dstv, idxv, srcv_t, dstv_t, idxv_t, rowsv, rowsv_t,
            acc, sem):
        c = lax.axis_index("c")
        s = lax.axis_index("s")
        r0 = s * ROWS_PER_TILE
        row_off = c * NP

        pltpu.sync_copy(h_hbm.at[pl.ds(row_off + r0, ROWS_PER_TILE)],
                        acc.at[pl.ds(r0, ROWS_PER_TILE)])
        plsc.subcore_barrier()

        ebase = s * EDGES_PER_TILE_AGG

        @pl.loop(0, AGG_FULL_CHUNKS)
        def _(j):
            b = pl.multiple_of(ebase + j * CHUNK, 8)
            pltpu.sync_copy(src_hbm.at[pl.ds(b, CHUNK)], srcv)
            pltpu.sync_copy(dst_hbm.at[pl.ds(b, CHUNK)], dstv)
            for i in range(CHUNK // 16):
                idxv[pl.ds(i * 16, 16)] = srcv[pl.ds(i * 16, 16)] + row_off
            pltpu.async_copy(h_hbm.at[idxv], rowsv, sem).wait()
            pltpu.sync_copy(rowsv, acc.at[dstv], add=True)

        bt = pl.multiple_of(ebase + AGG_FULL_CHUNKS * CHUNK, 8)
        pltpu.sync_copy(src_hbm.at[pl.ds(bt, AGG_TAIL)], srcv_t)
        pltpu.sync_copy(dst_hbm.at[pl.ds(bt, AGG_TAIL)], dstv_t)
        idxv_t[...] = srcv_t[...] + row_off
        pltpu.async_copy(h_hbm.at[idxv_t], rowsv_t, sem).wait()
        pltpu.sync_copy(rowsv_t, acc.at[dstv_t], add=True)

        plsc.subcore_barrier()
        pltpu.sync_copy(acc.at[pl.ds(r0, ROWS_PER_TILE)],
                        out_hbm.at[pl.ds(row_off + r0, ROWS_PER_TILE)])

    return agg


_sc_agg128 = _make_sc_agg(128)
_sc_agg32 = _make_sc_agg(32)


def _split_store(out_ref, h):
    hw = h.shape[1] // 2
    out_ref[0:N, :] = h[:, 0:hw]
    out_ref[NP:NP + N, :] = h[:, hw:]
    pad = jnp.zeros((NP - N, hw), jnp.float32)
    out_ref[N:NP, :] = pad
    out_ref[NP + N:2 * NP, :] = pad


def _dinv(degp_ref):
    deg = degp_ref[0, 0:N, 0:1] + degp_ref[1, 0:N, 0:1] + 1.0
    return lax.rsqrt(deg)


def _tc1_body(degp_ref, x_ref, w_ref, out_ref):
    dinv = _dinv(degp_ref)
    h = jnp.dot(x_ref[...], w_ref[...], preferred_element_type=jnp.float32)
    h = h * dinv
    _split_store(out_ref, h)


def _tc_mid_body(degp_ref, agg_ref, b_ref, g_ref, be_ref, w_ref, out_ref):
    dinv = _dinv(degp_ref)
    z = jnp.concatenate([agg_ref[0:N, :], agg_ref[NP:NP + N, :]], axis=1)
    z = z * dinv + b_ref[...]
    z = jnp.maximum(z, 0.0)
    mean = jnp.mean(z, axis=0, keepdims=True)
    var = jnp.mean((z - mean) * (z - mean), axis=0, keepdims=True)
    z = g_ref[...] * (z - mean) * lax.rsqrt(var + 1e-5) + be_ref[...]
    h = jnp.dot(z, w_ref[...], preferred_element_type=jnp.float32)
    h = h * dinv
    _split_store(out_ref, h)


def _tc_final_body(degp_ref, agg_ref, b_ref, out_ref):
    dinv = _dinv(degp_ref)
    z = jnp.concatenate([agg_ref[0:N, :], agg_ref[NP:NP + N, :]], axis=1)
    z = z * dinv + b_ref[...]
    m = jnp.max(z, axis=1, keepdims=True)
    zm = z - m
    lse = jnp.log(jnp.sum(jnp.exp(zm), axis=1, keepdims=True))
    out_ref[...] = zm - lse


def _tc_call(body, out_shape, *args):
    return pl.pallas_call(
        body, out_shape=jax.ShapeDtypeStruct(out_shape, jnp.float32))(*args)


def _agg_jnp(h, src, dst, HD):
    full = jnp.concatenate([h[0:N], h[NP:NP + N]], axis=1)
    agg = full.at[dst].add(full[src])
    out = jnp.zeros((2 * NP, HD), jnp.float32)
    out = out.at[0:N].set(agg[:, 0:HD]).at[NP:NP + N].set(agg[:, HD:])
    return out


def _deg_jnp(dst):
    deg = jnp.zeros((N,), jnp.float32).at[dst].add(1.0)
    return jnp.zeros((NC, NP, 16), jnp.float32).at[0, 0:N, :].set(deg[:, None])


def kernel(x, edge_index, W1, b1, W2, b2, W3, b3, gamma1, beta1, gamma2, beta2):
    src = edge_index[0]
    dst = edge_index[1]

    ones16 = jnp.ones((CHUNK, 16), jnp.float32)
    zeros16 = jnp.zeros((ROWS_PER_TILE, 16), jnp.float32)
    degp = _sc_degree(dst, ones16, zeros16)
    h1 = _tc_call(_tc1_body, (2 * NP, 128), degp, x, W1)
    a1 = _sc_agg128(h1, src, dst)
    h2 = _tc_call(_tc_mid_body, (2 * NP, 128), degp, a1,
                  b1.reshape(1, -1), gamma1.reshape(1, -1),
                  beta1.reshape(1, -1), W2)
    a2 = _sc_agg128(h2, src, dst)
    h3 = _tc_call(_tc_mid_body, (2 * NP, 32), degp, a2,
                  b2.reshape(1, -1), gamma2.reshape(1, -1),
                  beta2.reshape(1, -1), W3)
    a3 = _sc_agg32(h3, src, dst)
    out = _tc_call(_tc_final_body, (N, 64), degp, a3, b3.reshape(1, -1))
    return out
